# baseline (device time: 23272 ns/iter reference)
import jax
import jax.numpy as jnp
from jax import lax
from jax.experimental import pallas as pl
from jax.experimental.pallas import tpu as pltpu

N_DEV = 4
B_LOC = 2
SQ = 256
SKV = 256
HQ = 16
HQ_LOC = 4
DH = 64
D = 512
DHID = 256
BLK = 64

SLOT_ORDER = (0, 3, 1, 2)


def _body(x_ref, wq_ref, k_ref, v_ref, wo_ref, out_ref,
          cwq, cwo, ctx_ref, xb_ref, swq, rwq, swo, rwo):
    my = lax.axis_index("i")

    xb_ref[...] = x_ref[...].reshape(B_LOC * SQ, D).astype(jnp.bfloat16)
    cwq[0] = wq_ref[...].astype(jnp.bfloat16)
    cwo[0] = wo_ref[...].astype(jnp.bfloat16)

    barrier = pltpu.get_barrier_semaphore()
    for d in range(1, N_DEV):
        peer = lax.rem(my + d, N_DEV)
        pl.semaphore_signal(barrier, inc=1, device_id=(peer,),
                            device_id_type=pl.DeviceIdType.MESH)
    pl.semaphore_wait(barrier, N_DEV - 1)

    def bcast(comm, ssem, rsem, d):
        peer = lax.rem(my + d, N_DEV)
        r = pltpu.make_async_remote_copy(
            src_ref=comm.at[0], dst_ref=comm.at[N_DEV - d],
            send_sem=ssem.at[d - 1], recv_sem=rsem.at[N_DEV - d],
            device_id=(peer,), device_id_type=pl.DeviceIdType.MESH)
        r.start()
        return r

    def recv_wait(comm, ssem, rsem, s):
        pltpu.make_async_remote_copy(
            src_ref=comm.at[s], dst_ref=comm.at[s],
            send_sem=ssem.at[0], recv_sem=rsem.at[s],
            device_id=(my,), device_id_type=pl.DeviceIdType.MESH,
        ).wait_recv()

    sends = [bcast(cwq, swq, rwq, d) for d in (1, 3, 2)]

    qb = lax.broadcasted_iota(jnp.int32, (SQ, SKV), 0) // BLK
    kb = lax.broadcasted_iota(jnp.int32, (SQ, SKV), 1) // BLK
    maskb = (qb == kb) | (kb == 0) | (lax.rem(qb + kb, 3) == 0)
    maskf = jnp.where(maskb, jnp.float32(1.0), jnp.float32(0.0))

    def attention(s):
        q2 = jnp.dot(xb_ref[...], cwq[s], preferred_element_type=jnp.float32)
        q2 = (q2 * jnp.float32(0.125)).astype(jnp.bfloat16)
        for b in range(B_LOC):
            for hl in range(HQ_LOC):
                c0 = s * DHID + hl * DH
                kk = k_ref[b, :, c0:c0 + DH]
                vv = v_ref[b, :, c0:c0 + DH]
                qq = q2[b * SQ:(b + 1) * SQ, hl * DH:(hl + 1) * DH]
                sc = lax.dot_general(qq, kk, (((1,), (1,)), ((), ())),
                                     preferred_element_type=jnp.float32)
                w = jnp.exp(sc) * maskf
                r = jnp.sum(w, axis=1, keepdims=True)
                cx = jnp.dot(w.astype(jnp.bfloat16), vv,
                             preferred_element_type=jnp.float32)
                cx = cx * (jnp.float32(1.0) / r)
                ctx_ref[b * SQ:(b + 1) * SQ,
                        c0:c0 + DH] = cx.astype(jnp.bfloat16)

    attention(0)
    sends += [bcast(cwo, swo, rwo, d) for d in (1, 3, 2)]
    for s in SLOT_ORDER[1:]:
        recv_wait(cwq, swq, rwq, s)
        attention(s)

    for s in SLOT_ORDER[1:]:
        recv_wait(cwo, swo, rwo, s)
    out = jnp.dot(ctx_ref[...], cwo[...].reshape(N_DEV * DHID, D),
                  preferred_element_type=jnp.float32)
    out_ref[...] = out.reshape(B_LOC, SQ, D)

    for r in sends:
        r.wait_send()


def kernel(x, Wq, K_ext, V_ext, Wo):
    my = lax.axis_index("i")
    def prep(a):
        a = a.reshape(N_DEV * B_LOC, SKV, HQ * DH)
        a = lax.dynamic_slice(
            a, (my * B_LOC, 0, 0), (B_LOC, SKV, HQ * DH))
        return jnp.roll(a, -my * DHID, axis=2).astype(jnp.bfloat16)

    kv = prep(K_ext)
    vv = prep(V_ext)

    return pl.pallas_call(
        _body,
        out_shape=jax.ShapeDtypeStruct((B_LOC, SQ, D), jnp.float32),
        in_specs=[pl.BlockSpec(memory_space=pltpu.VMEM)] * 5,
        out_specs=pl.BlockSpec(memory_space=pltpu.VMEM),
        scratch_shapes=[
            pltpu.VMEM((N_DEV, D, DHID), jnp.bfloat16),
            pltpu.VMEM((N_DEV, DHID, D), jnp.bfloat16),
            pltpu.VMEM((B_LOC * SQ, N_DEV * DHID), jnp.bfloat16),
            pltpu.VMEM((B_LOC * SQ, D), jnp.bfloat16),
            pltpu.SemaphoreType.DMA((N_DEV - 1,)),
            pltpu.SemaphoreType.DMA((N_DEV,)),
            pltpu.SemaphoreType.DMA((N_DEV - 1,)),
            pltpu.SemaphoreType.DMA((N_DEV,)),
        ],
        compiler_params=pltpu.CompilerParams(collective_id=0),
    )(x, Wq, kv, vv, Wo)


# device time: 23172 ns/iter; 1.0043x vs baseline; 1.0043x over previous
import jax
import jax.numpy as jnp
from jax import lax
from jax.experimental import pallas as pl
from jax.experimental.pallas import tpu as pltpu

N_DEV = 4
B_LOC = 2
SQ = 256
SKV = 256
HQ = 16
HQ_LOC = 4
DH = 64
D = 512
DHID = 256
BLK = 64

SLOT_ORDER = (0, 3, 1, 2)


def _body(x_ref, wq_ref, k_ref, v_ref, wo_ref, out_ref,
          cwq, cwo, ctx_ref, xb_ref, swq, rwq, swo, rwo):
    my = lax.axis_index("i")

    xb_ref[...] = x_ref[...].reshape(B_LOC * SQ, D).astype(jnp.bfloat16)
    cwq[0] = wq_ref[...].astype(jnp.bfloat16)
    cwo[0] = wo_ref[...].astype(jnp.bfloat16)

    barrier = pltpu.get_barrier_semaphore()
    for d in range(1, N_DEV):
        peer = lax.rem(my + d, N_DEV)
        pl.semaphore_signal(barrier, inc=1, device_id=(peer,),
                            device_id_type=pl.DeviceIdType.MESH)
    pl.semaphore_wait(barrier, N_DEV - 1)

    def bcast(comm, ssem, rsem, d):
        peer = lax.rem(my + d, N_DEV)
        r = pltpu.make_async_remote_copy(
            src_ref=comm.at[0], dst_ref=comm.at[N_DEV - d],
            send_sem=ssem.at[d - 1], recv_sem=rsem.at[N_DEV - d],
            device_id=(peer,), device_id_type=pl.DeviceIdType.MESH)
        r.start()
        return r

    def recv_wait(comm, ssem, rsem, s):
        pltpu.make_async_remote_copy(
            src_ref=comm.at[s], dst_ref=comm.at[s],
            send_sem=ssem.at[0], recv_sem=rsem.at[s],
            device_id=(my,), device_id_type=pl.DeviceIdType.MESH,
        ).wait_recv()

    sends = [bcast(cwq, swq, rwq, d) for d in (1, 3, 2)]

    qb = lax.broadcasted_iota(jnp.int32, (SQ, SKV), 0) // BLK
    kb = lax.broadcasted_iota(jnp.int32, (SQ, SKV), 1) // BLK
    maskb = (qb == kb) | (kb == 0) | (lax.rem(qb + kb, 3) == 0)
    maskf = jnp.where(maskb, jnp.float32(1.0), jnp.float32(0.0))

    def attention(s):
        q2 = jnp.dot(xb_ref[...], cwq[s], preferred_element_type=jnp.float32)
        q2 = (q2 * jnp.float32(0.125)).astype(jnp.bfloat16)
        for b in range(B_LOC):
            for hl in range(HQ_LOC):
                c0 = s * DHID + hl * DH
                kk = k_ref[b, :, c0:c0 + DH]
                vv = v_ref[b, :, c0:c0 + DH]
                qq = q2[b * SQ:(b + 1) * SQ, hl * DH:(hl + 1) * DH]
                sc = lax.dot_general(qq, kk, (((1,), (1,)), ((), ())),
                                     preferred_element_type=jnp.float32)
                w = jnp.exp(sc) * maskf
                r = jnp.sum(w, axis=1, keepdims=True)
                cx = jnp.dot(w.astype(jnp.bfloat16), vv,
                             preferred_element_type=jnp.float32)
                cx = cx * (jnp.float32(1.0) / r)
                ctx_ref[b * SQ:(b + 1) * SQ,
                        c0:c0 + DH] = cx.astype(jnp.bfloat16)

    attention(0)
    sends += [bcast(cwo, swo, rwo, d) for d in (1, 3, 2)]
    for s in SLOT_ORDER[1:]:
        recv_wait(cwq, swq, rwq, s)
        attention(s)

    for s in SLOT_ORDER[1:]:
        recv_wait(cwo, swo, rwo, s)
    out = jnp.dot(ctx_ref[...], cwo[...].reshape(N_DEV * DHID, D),
                  preferred_element_type=jnp.float32)
    out_ref[...] = out.reshape(B_LOC, SQ, D)

    for r in sends:
        r.wait_send()


def kernel(x, Wq, K_ext, V_ext, Wo):
    my = lax.axis_index("i")
    def prep(a):
        a = a.reshape(N_DEV * B_LOC, SKV, HQ * DH)
        parts = []
        for s in range(N_DEV):
            g = lax.rem(my + s, N_DEV)
            parts.append(lax.dynamic_slice(
                a, (my * B_LOC, 0, g * DHID), (B_LOC, SKV, DHID)))
        return jnp.concatenate(parts, axis=2).astype(jnp.bfloat16)

    kv = prep(K_ext)
    vv = prep(V_ext)

    return pl.pallas_call(
        _body,
        out_shape=jax.ShapeDtypeStruct((B_LOC, SQ, D), jnp.float32),
        in_specs=[pl.BlockSpec(memory_space=pltpu.VMEM)] * 5,
        out_specs=pl.BlockSpec(memory_space=pltpu.VMEM),
        scratch_shapes=[
            pltpu.VMEM((N_DEV, D, DHID), jnp.bfloat16),
            pltpu.VMEM((N_DEV, DHID, D), jnp.bfloat16),
            pltpu.VMEM((B_LOC * SQ, N_DEV * DHID), jnp.bfloat16),
            pltpu.VMEM((B_LOC * SQ, D), jnp.bfloat16),
            pltpu.SemaphoreType.DMA((N_DEV - 1,)),
            pltpu.SemaphoreType.DMA((N_DEV,)),
            pltpu.SemaphoreType.DMA((N_DEV - 1,)),
            pltpu.SemaphoreType.DMA((N_DEV,)),
        ],
        compiler_params=pltpu.CompilerParams(collective_id=0),
    )(x, Wq, kv, vv, Wo)


# device time: 21783 ns/iter; 1.0684x vs baseline; 1.0638x over previous
import jax
import jax.numpy as jnp
from jax import lax
from jax.experimental import pallas as pl
from jax.experimental.pallas import tpu as pltpu

N_DEV = 4
B_LOC = 2
SQ = 256
SKV = 256
HQ = 16
HQ_LOC = 4
DH = 64
D = 512
DHID = 256
BLK = 64

SLOT_ORDER = (0, 3, 1, 2)


def _body(x_ref, wq_ref, k0, k1, k2, k3, v0, v1, v2, v3, wo_ref, out_ref,
          cwq, cwo, ctx_ref, xb_ref, swq, rwq, swo, rwo):
    k_refs = (k0, k1, k2, k3)
    v_refs = (v0, v1, v2, v3)
    my = lax.axis_index("i")

    xb_ref[...] = x_ref[...].reshape(B_LOC * SQ, D).astype(jnp.bfloat16)
    cwq[0] = wq_ref[...].astype(jnp.bfloat16)
    cwo[0] = wo_ref[...].astype(jnp.bfloat16)

    barrier = pltpu.get_barrier_semaphore()
    for d in range(1, N_DEV):
        peer = lax.rem(my + d, N_DEV)
        pl.semaphore_signal(barrier, inc=1, device_id=(peer,),
                            device_id_type=pl.DeviceIdType.MESH)
    pl.semaphore_wait(barrier, N_DEV - 1)

    def bcast(comm, ssem, rsem, d):
        peer = lax.rem(my + d, N_DEV)
        r = pltpu.make_async_remote_copy(
            src_ref=comm.at[0], dst_ref=comm.at[N_DEV - d],
            send_sem=ssem.at[d - 1], recv_sem=rsem.at[N_DEV - d],
            device_id=(peer,), device_id_type=pl.DeviceIdType.MESH)
        r.start()
        return r

    def recv_wait(comm, ssem, rsem, s):
        pltpu.make_async_remote_copy(
            src_ref=comm.at[s], dst_ref=comm.at[s],
            send_sem=ssem.at[0], recv_sem=rsem.at[s],
            device_id=(my,), device_id_type=pl.DeviceIdType.MESH,
        ).wait_recv()

    sends = [bcast(cwq, swq, rwq, d) for d in (1, 3, 2)]

    qb = lax.broadcasted_iota(jnp.int32, (SQ, SKV), 0) // BLK
    kb = lax.broadcasted_iota(jnp.int32, (SQ, SKV), 1) // BLK
    maskb = (qb == kb) | (kb == 0) | (lax.rem(qb + kb, 3) == 0)
    maskf = jnp.where(maskb, jnp.float32(1.0), jnp.float32(0.0))

    def attention(s):
        q2 = jnp.dot(xb_ref[...], cwq[s], preferred_element_type=jnp.float32)
        q2 = (q2 * jnp.float32(0.125)).astype(jnp.bfloat16)
        for b in range(B_LOC):
            for hl in range(HQ_LOC):
                c0 = s * DHID + hl * DH
                kk = k_refs[s][b, :, hl * DH:(hl + 1) * DH]
                vv = v_refs[s][b, :, hl * DH:(hl + 1) * DH]
                qq = q2[b * SQ:(b + 1) * SQ, hl * DH:(hl + 1) * DH]
                sc = lax.dot_general(qq, kk, (((1,), (1,)), ((), ())),
                                     preferred_element_type=jnp.float32)
                w = jnp.exp(sc) * maskf
                r = jnp.sum(w, axis=1, keepdims=True)
                cx = jnp.dot(w.astype(jnp.bfloat16), vv,
                             preferred_element_type=jnp.float32)
                cx = cx * (jnp.float32(1.0) / r)
                ctx_ref[b * SQ:(b + 1) * SQ,
                        c0:c0 + DH] = cx.astype(jnp.bfloat16)

    attention(0)
    sends += [bcast(cwo, swo, rwo, d) for d in (1, 3, 2)]
    for s in SLOT_ORDER[1:]:
        recv_wait(cwq, swq, rwq, s)
        attention(s)

    for t, s in enumerate(SLOT_ORDER):
        if s != 0:
            recv_wait(cwo, swo, rwo, s)
        contrib = jnp.dot(ctx_ref[:, s * DHID:(s + 1) * DHID], cwo[s],
                          preferred_element_type=jnp.float32)
        if t == 0:
            out_ref[...] = contrib.reshape(B_LOC, SQ, D)
        else:
            out_ref[...] = out_ref[...] + contrib.reshape(B_LOC, SQ, D)

    for r in sends:
        r.wait_send()


def kernel(x, Wq, K_ext, V_ext, Wo):
    my = lax.axis_index("i")
    def prep(a):
        a = a.reshape(N_DEV * B_LOC, SKV, HQ * DH)
        parts = []
        for s in range(N_DEV):
            g = lax.rem(my + s, N_DEV)
            parts.append(lax.dynamic_slice(
                a, (my * B_LOC, 0, g * DHID),
                (B_LOC, SKV, DHID)).astype(jnp.bfloat16))
        return parts

    kv = prep(K_ext)
    vv = prep(V_ext)

    return pl.pallas_call(
        _body,
        out_shape=jax.ShapeDtypeStruct((B_LOC, SQ, D), jnp.float32),
        in_specs=[pl.BlockSpec(memory_space=pltpu.VMEM)] * 11,
        out_specs=pl.BlockSpec(memory_space=pltpu.VMEM),
        scratch_shapes=[
            pltpu.VMEM((N_DEV, D, DHID), jnp.bfloat16),
            pltpu.VMEM((N_DEV, DHID, D), jnp.bfloat16),
            pltpu.VMEM((B_LOC * SQ, N_DEV * DHID), jnp.bfloat16),
            pltpu.VMEM((B_LOC * SQ, D), jnp.bfloat16),
            pltpu.SemaphoreType.DMA((N_DEV - 1,)),
            pltpu.SemaphoreType.DMA((N_DEV,)),
            pltpu.SemaphoreType.DMA((N_DEV - 1,)),
            pltpu.SemaphoreType.DMA((N_DEV,)),
        ],
        compiler_params=pltpu.CompilerParams(collective_id=0),
    )(x, Wq, *kv, *vv, Wo)


# device time: 21733 ns/iter; 1.0708x vs baseline; 1.0023x over previous
import jax
import jax.numpy as jnp
from jax import lax
from jax.experimental import pallas as pl
from jax.experimental.pallas import tpu as pltpu

N_DEV = 4
B_LOC = 2
SQ = 256
SKV = 256
HQ = 16
HQ_LOC = 4
DH = 64
D = 512
DHID = 256
BLK = 64

SLOT_ORDER = (0, 3, 1, 2)


def _body(x_ref, wq_ref, k0, k1, k2, k3, v0, v1, v2, v3, wo_ref, out_ref,
          cwq, cwo, ctx_ref, xb_ref, swq, rwq, swo, rwo):
    k_refs = (k0, k1, k2, k3)
    v_refs = (v0, v1, v2, v3)
    my = lax.axis_index("i")

    barrier = pltpu.get_barrier_semaphore()
    for d in range(1, N_DEV):
        peer = lax.rem(my + d, N_DEV)
        pl.semaphore_signal(barrier, inc=1, device_id=(peer,),
                            device_id_type=pl.DeviceIdType.MESH)

    xb_ref[...] = x_ref[...].reshape(B_LOC * SQ, D).astype(jnp.bfloat16)
    cwq[0] = wq_ref[...].astype(jnp.bfloat16)
    cwo[0] = wo_ref[...].astype(jnp.bfloat16)

    pl.semaphore_wait(barrier, N_DEV - 1)

    def bcast(comm, ssem, rsem, d):
        peer = lax.rem(my + d, N_DEV)
        r = pltpu.make_async_remote_copy(
            src_ref=comm.at[0], dst_ref=comm.at[N_DEV - d],
            send_sem=ssem.at[d - 1], recv_sem=rsem.at[N_DEV - d],
            device_id=(peer,), device_id_type=pl.DeviceIdType.MESH)
        r.start()
        return r

    def recv_wait(comm, ssem, rsem, s):
        pltpu.make_async_remote_copy(
            src_ref=comm.at[s], dst_ref=comm.at[s],
            send_sem=ssem.at[0], recv_sem=rsem.at[s],
            device_id=(my,), device_id_type=pl.DeviceIdType.MESH,
        ).wait_recv()

    sends = [bcast(cwq, swq, rwq, d) for d in (1, 3, 2)]

    qb = lax.broadcasted_iota(jnp.int32, (SQ, SKV), 0) // BLK
    kb = lax.broadcasted_iota(jnp.int32, (SQ, SKV), 1) // BLK
    maskb = (qb == kb) | (kb == 0) | (lax.rem(qb + kb, 3) == 0)
    maskf = jnp.where(maskb, jnp.float32(1.0), jnp.float32(0.0))

    def attention(s):
        q2 = jnp.dot(xb_ref[...], cwq[s], preferred_element_type=jnp.float32)
        q2 = (q2 * jnp.float32(0.125)).astype(jnp.bfloat16)
        for b in range(B_LOC):
            for hl in range(HQ_LOC):
                c0 = s * DHID + hl * DH
                kk = k_refs[s][b, :, hl * DH:(hl + 1) * DH]
                vv = v_refs[s][b, :, hl * DH:(hl + 1) * DH]
                qq = q2[b * SQ:(b + 1) * SQ, hl * DH:(hl + 1) * DH]
                sc = lax.dot_general(qq, kk, (((1,), (1,)), ((), ())),
                                     preferred_element_type=jnp.float32)
                w = jnp.exp(sc) * maskf
                r = jnp.sum(w, axis=1, keepdims=True)
                cx = jnp.dot(w.astype(jnp.bfloat16), vv,
                             preferred_element_type=jnp.float32)
                cx = cx * (jnp.float32(1.0) / r)
                ctx_ref[b * SQ:(b + 1) * SQ,
                        c0:c0 + DH] = cx.astype(jnp.bfloat16)

    attention(0)
    sends += [bcast(cwo, swo, rwo, d) for d in (1, 3, 2)]
    for s in SLOT_ORDER[1:]:
        recv_wait(cwq, swq, rwq, s)
        attention(s)

    for t, s in enumerate(SLOT_ORDER):
        if s != 0:
            recv_wait(cwo, swo, rwo, s)
        contrib = jnp.dot(ctx_ref[:, s * DHID:(s + 1) * DHID], cwo[s],
                          preferred_element_type=jnp.float32)
        if t == 0:
            out_ref[...] = contrib.reshape(B_LOC, SQ, D)
        else:
            out_ref[...] = out_ref[...] + contrib.reshape(B_LOC, SQ, D)

    for r in sends:
        r.wait_send()


def kernel(x, Wq, K_ext, V_ext, Wo):
    my = lax.axis_index("i")
    def prep(a):
        a = a.reshape(N_DEV * B_LOC, SKV, HQ * DH)
        parts = []
        for s in range(N_DEV):
            g = lax.rem(my + s, N_DEV)
            parts.append(lax.dynamic_slice(
                a, (my * B_LOC, 0, g * DHID),
                (B_LOC, SKV, DHID)).astype(jnp.bfloat16))
        return parts

    kv = prep(K_ext)
    vv = prep(V_ext)

    return pl.pallas_call(
        _body,
        out_shape=jax.ShapeDtypeStruct((B_LOC, SQ, D), jnp.float32),
        in_specs=[pl.BlockSpec(memory_space=pltpu.VMEM)] * 11,
        out_specs=pl.BlockSpec(memory_space=pltpu.VMEM),
        scratch_shapes=[
            pltpu.VMEM((N_DEV, D, DHID), jnp.bfloat16),
            pltpu.VMEM((N_DEV, DHID, D), jnp.bfloat16),
            pltpu.VMEM((B_LOC * SQ, N_DEV * DHID), jnp.bfloat16),
            pltpu.VMEM((B_LOC * SQ, D), jnp.bfloat16),
            pltpu.SemaphoreType.DMA((N_DEV - 1,)),
            pltpu.SemaphoreType.DMA((N_DEV,)),
            pltpu.SemaphoreType.DMA((N_DEV - 1,)),
            pltpu.SemaphoreType.DMA((N_DEV,)),
        ],
        compiler_params=pltpu.CompilerParams(collective_id=0),
    )(x, Wq, *kv, *vv, Wo)
